# baseline (speedup 1.0000x reference)
import functools

import jax
import jax.numpy as jnp
from jax import lax
from jax.experimental import pallas as pl
from jax.experimental.pallas import tpu as pltpu
from jax.experimental.pallas import tpu_sc as plsc

N_NODES = 10000
N_EDGES = 320000
D = 128
NCLS = 16
N_GRAPHS = 64

NC, NS = 2, 16
NW = NC * NS
NPAD = 10240
RPW = NPAD // NS
EP = 327680
CHW = 32
CHN = 128
NBUF = 4

BR = 1024
GRID = NPAD // BR


def _sc_agg(table, src, dst, zeros_tbl, zeros16, ones16, chunk, nbuf,
            with_cnt):
    width = table.shape[1]
    nt = EP // (NW * chunk)
    mesh = plsc.VectorSubcoreMesh(core_axis_name="c", subcore_axis_name="s")

    out_type = [jax.ShapeDtypeStruct((NC, NPAD, width), jnp.float32)]
    scratch = [
        pltpu.VMEM((nt, chunk), jnp.int32),
        pltpu.VMEM((nt, chunk), jnp.int32),
    ]
    scratch += [pltpu.VMEM((chunk, width), jnp.float32) for _ in range(nbuf)]
    scratch += [pltpu.VMEM_SHARED((NPAD, width), jnp.float32)]
    if with_cnt:
        out_type.append(jax.ShapeDtypeStruct((NC, NPAD, NCLS), jnp.float32))
        scratch += [pltpu.VMEM((chunk, NCLS), jnp.float32),
                    pltpu.VMEM_SHARED((NPAD, NCLS), jnp.float32)]
    scratch += [pltpu.SemaphoreType.DMA for _ in range(2 * nbuf)]

    @functools.partial(
        pl.kernel,
        out_type=tuple(out_type) if with_cnt else out_type[0],
        mesh=mesh,
        compiler_params=pltpu.CompilerParams(use_tc_tiling_on_sc=False),
        scratch_types=scratch,
    )
    def k(tbl_hbm, src_hbm, dst_hbm, ztbl_hbm, *rest):
        if with_cnt:
            (z16_hbm, ones_hbm, acc_out, cnt_out, sib, dib, *bufs) = rest
            rows = bufs[:nbuf]
            acc_sh, ones, cnt_sh = bufs[nbuf], bufs[nbuf + 1], bufs[nbuf + 2]
            sems = bufs[nbuf + 3:]
        else:
            (acc_out, sib, dib, *bufs) = rest
            rows = bufs[:nbuf]
            acc_sh = bufs[nbuf]
            sems = bufs[nbuf + 1:]
        sem_g, sem_s = sems[:nbuf], sems[nbuf:]

        cid = lax.axis_index("c")
        sid = lax.axis_index("s")
        wid = cid * NS + sid

        r0 = sid * RPW
        pltpu.sync_copy(ztbl_hbm.at[pl.ds(r0, RPW), :],
                        acc_sh.at[pl.ds(r0, RPW), :])
        if with_cnt:
            pltpu.sync_copy(z16_hbm.at[pl.ds(r0, RPW), :],
                            cnt_sh.at[pl.ds(r0, RPW), :])
            pltpu.sync_copy(ones_hbm, ones)

        w0 = wid * nt
        pltpu.sync_copy(src_hbm.at[pl.ds(w0, nt), :], sib)
        pltpu.sync_copy(dst_hbm.at[pl.ds(w0, nt), :], dib)

        plsc.subcore_barrier()

        def start_gather(b, c):
            pltpu.async_copy(tbl_hbm.at[sib.at[c]], rows[b], sem_g[b])

        def wait_gather(b, c):
            pltpu.make_async_copy(tbl_hbm.at[sib.at[c]], rows[b],
                                  sem_g[b]).wait()

        def start_scatter(b, c):
            pltpu.async_copy(rows[b], acc_sh.at[dib.at[c]], sem_s[b],
                             add=True)
            if with_cnt:
                pltpu.async_copy(ones, cnt_sh.at[dib.at[c]], sem_s[b],
                                 add=True)

        def wait_scatter(b, c):
            pltpu.make_async_copy(rows[b], acc_sh.at[dib.at[c]],
                                  sem_s[b]).wait()
            if with_cnt:
                pltpu.make_async_copy(ones, cnt_sh.at[dib.at[c]],
                                      sem_s[b]).wait()

        for b in range(nbuf):
            start_gather(b, b)

        @pl.loop(0, nt - nbuf, step=nbuf)
        def _(t):
            for b in range(nbuf):
                wait_gather(b, t + b)
                start_scatter(b, t + b)
            for b in range(nbuf):
                wait_scatter(b, t + b)
                start_gather(b, t + b + nbuf)

        for b in range(nbuf):
            wait_gather(b, nt - nbuf + b)
            start_scatter(b, nt - nbuf + b)
        for b in range(nbuf):
            wait_scatter(b, nt - nbuf + b)

        plsc.subcore_barrier()

        pltpu.sync_copy(acc_sh.at[pl.ds(r0, RPW), :],
                        acc_out.at[cid, pl.ds(r0, RPW), :])
        if with_cnt:
            pltpu.sync_copy(cnt_sh.at[pl.ds(r0, RPW), :],
                            cnt_out.at[cid, pl.ds(r0, RPW), :])

    if with_cnt:
        return k(table, src, dst, zeros_tbl, zeros16, ones16)
    return k(table, src, dst, zeros_tbl)


def _tc_layer1(x_pad, s1a, s1b, cnta, cntb, Wl1, bl1, Wr1, Wl2):
    def body(x_ref, sa_ref, sb_ref, ca_ref, cb_ref, wl1_ref, bl1_ref,
             wr1_ref, wl2_ref, h1_ref, hl_ref, inv_ref):
        cnt = ca_ref[:, 0:1] + cb_ref[:, 0:1]
        inv = 1.0 / jnp.maximum(cnt, 1.0)
        inv_ref[...] = inv
        s1 = (sa_ref[...] + sb_ref[...]) * inv
        h1 = s1 @ wl1_ref[...] + bl1_ref[...] + x_ref[...] @ wr1_ref[...]
        h1 = jnp.maximum(h1, 0.0)
        h1_ref[...] = h1
        hl_ref[...] = h1 @ wl2_ref[...]

    row_spec = pl.BlockSpec((BR, D), lambda i: (i, 0))
    nar_spec = pl.BlockSpec((BR, NCLS), lambda i: (i, 0))
    col_spec = pl.BlockSpec((BR, 1), lambda i: (i, 0))
    full = lambda shape: pl.BlockSpec(shape, lambda i: tuple(0 for _ in shape))
    return pl.pallas_call(
        body,
        grid=(GRID,),
        in_specs=[row_spec, row_spec, row_spec, nar_spec, nar_spec,
                  full((D, D)), full((1, D)), full((D, D)), full((D, NCLS))],
        out_specs=[row_spec, pl.BlockSpec((BR, NCLS), lambda i: (i, 0)),
                   col_spec],
        out_shape=[
            jax.ShapeDtypeStruct((NPAD, D), jnp.float32),
            jax.ShapeDtypeStruct((NPAD, NCLS), jnp.float32),
            jax.ShapeDtypeStruct((NPAD, 1), jnp.float32),
        ],
    )(x_pad, s1a, s1b, cnta, cntb, Wl1, bl1.reshape(1, D), Wr1, Wl2)


def _tc_layer2(h1, s2a, s2b, inv, Wr2, bl2, batch_col):
    def body(h1_ref, sa_ref, sb_ref, inv_ref, wr2_ref, bl2_ref, b_ref,
             out_ref, psum, pcnt):
        i = pl.program_id(0)

        @pl.when(i == 0)
        def _():
            psum[...] = jnp.zeros_like(psum)
            pcnt[...] = jnp.zeros_like(pcnt)

        h2 = (sa_ref[...] + sb_ref[...]) * inv_ref[...] + bl2_ref[...] \
            + h1_ref[...] @ wr2_ref[...]
        oh = (b_ref[...] == lax.broadcasted_iota(jnp.int32, (1, N_GRAPHS), 1))
        oh = oh.astype(jnp.float32)
        dn = (((0,), (0,)), ((), ()))
        psum[...] += lax.dot_general(oh, h2, dn,
                                     preferred_element_type=jnp.float32)
        pcnt[...] += lax.dot_general(oh, jnp.ones((BR, NCLS), jnp.float32), dn,
                                     preferred_element_type=jnp.float32)

        @pl.when(i == GRID - 1)
        def _():
            p = psum[...] / jnp.maximum(pcnt[...], 1.0)
            m = jnp.max(p, axis=1, keepdims=True)
            e = jnp.exp(p - m)
            lse = jnp.log(jnp.sum(e, axis=1, keepdims=True))
            out_ref[...] = p - m - lse

    row_spec = pl.BlockSpec((BR, D), lambda i: (i, 0))
    nar_spec = pl.BlockSpec((BR, NCLS), lambda i: (i, 0))
    col_spec = pl.BlockSpec((BR, 1), lambda i: (i, 0))
    full = lambda shape: pl.BlockSpec(shape, lambda i: tuple(0 for _ in shape))
    return pl.pallas_call(
        body,
        grid=(GRID,),
        in_specs=[row_spec, nar_spec, nar_spec, col_spec,
                  full((D, NCLS)), full((1, NCLS)), col_spec],
        out_specs=full((N_GRAPHS, NCLS)),
        out_shape=jax.ShapeDtypeStruct((N_GRAPHS, NCLS), jnp.float32),
        scratch_shapes=[
            pltpu.VMEM((N_GRAPHS, NCLS), jnp.float32),
            pltpu.VMEM((N_GRAPHS, NCLS), jnp.float32),
        ],
    )(h1, s2a, s2b, inv, Wr2, bl2.reshape(1, NCLS), batch_col)


def kernel(x, edge_index, batch, Wl1, bl1, Wr1, Wl2, bl2, Wr2):
    npad_e = EP - N_EDGES
    pad_dst = (N_NODES + jnp.arange(npad_e, dtype=jnp.int32)
               % (NPAD - N_NODES))
    src = jnp.concatenate([edge_index[0], jnp.zeros((npad_e,), jnp.int32)])
    dst = jnp.concatenate([edge_index[1], pad_dst])
    zeros2d = jnp.zeros((NPAD, D), jnp.float32)
    zeros16 = jnp.zeros((NPAD, NCLS), jnp.float32)
    ones16 = jnp.ones((CHW, NCLS), jnp.float32)

    accs, cnts = _sc_agg(x, src.reshape(-1, CHW), dst.reshape(-1, CHW),
                         zeros2d, zeros16, ones16, CHW, NBUF, True)

    x_pad = jnp.concatenate([x, jnp.zeros((NPAD - N_NODES, D), jnp.float32)])
    h1, hl, inv = _tc_layer1(
        x_pad, accs[0], accs[1], cnts[0], cnts[1],
        Wl1, bl1, Wr1, Wl2)

    s2 = _sc_agg(hl, src.reshape(-1, CHN), dst.reshape(-1, CHN),
                 zeros16, None, None, CHN, NBUF, False)

    batch_col = jnp.concatenate(
        [batch, jnp.full((NPAD - N_NODES,), N_GRAPHS, jnp.int32)]
    ).reshape(NPAD, 1)
    return _tc_layer2(h1, s2[0], s2[1], inv, Wr2, bl2, batch_col)
